# baseline (speedup 1.0000x reference)
import functools

import jax
import jax.numpy as jnp
from jax import lax
from jax.experimental import pallas as pl
from jax.experimental.pallas import tpu as pltpu
from jax.experimental.pallas import tpu_sc as plsc

_EPS = 1e-07
_N = 2097152
_H = 512
_W = 512
_HW = _H * _W
_NT = 32
_NREG = 4
_G = _HW // _NREG
_NCHUNK = _NT // _NREG
_CHUNK = _N // _NCHUNK
_WIN = 4096
_NWIN = _CHUNK // _WIN
_PIX_T = _HW // _NT
_LANES = 16

_ROWS = _N // 128
_BLK = 1024


def _proj_body(x_ref, y_ref, z_ref, k_ref, lin_ref):
    x = x_ref[...]
    y = y_ref[...]
    z = z_ref[...]
    u = x * k_ref[0, 0] + y * k_ref[0, 1] + z * k_ref[0, 2]
    v = x * k_ref[1, 0] + y * k_ref[1, 1] + z * k_ref[1, 2]
    w = x * k_ref[2, 0] + y * k_ref[2, 1] + z * k_ref[2, 2]
    d = w + _EPS
    gx = u / d
    gy = v / d
    px = jnp.clip(jnp.floor(gx - 0.5), 0.0, float(_W - 1)).astype(jnp.int32)
    py = jnp.clip(jnp.floor(gy - 0.5), 0.0, float(_H - 1)).astype(jnp.int32)
    lin_ref[...] = py * _W + px


def _project(xp, yp, zp, intrinsics):
    spec = pl.BlockSpec((_BLK, 128), lambda i: (i, 0))
    return pl.pallas_call(
        _proj_body,
        grid=(_ROWS // _BLK,),
        in_specs=[spec, spec, spec,
                  pl.BlockSpec(memory_space=pltpu.SMEM)],
        out_specs=spec,
        out_shape=jax.ShapeDtypeStruct((_ROWS, 128), jnp.int32),
    )(xp, yp, zp, intrinsics)


@functools.cache
def _sc_mesh():
    return plsc.VectorSubcoreMesh(core_axis_name="c", subcore_axis_name="s",
                                  num_cores=2, num_subcores=16)


_UNROLL = 8


def _winner_body(lin_hbm, grids_hbm, grid_v, buf0_v, buf1_v, sem0, sem1):
    wid = lax.axis_index("s") * 2 + lax.axis_index("c")
    region = wid & 3
    chunk = wid >> 2
    lane = lax.iota(jnp.int32, _LANES)
    minus1 = jnp.full((_LANES,), -1, jnp.int32)

    @pl.loop(0, _G, step=_LANES, unroll=_UNROLL)
    def _init(i):
        grid_v[pl.ds(i, _LANES)] = minus1

    rbase = region * _G
    cbase = chunk * _CHUNK
    bufs = (buf0_v, buf1_v)
    sems = (sem0, sem1)

    pltpu.async_copy(lin_hbm.at[pl.ds(cbase, _WIN)], buf0_v, sem0)

    def _outer(g, carry):
        for b in range(2):
            w = 2 * g + b
            buf = bufs[b]
            pltpu.make_async_copy(
                lin_hbm.at[pl.ds(0, _WIN)], buf, sems[b]).wait()

            @pl.when(w < _NWIN - 1)
            def _start_next():
                pltpu.async_copy(
                    lin_hbm.at[pl.ds(cbase + (w + 1) * _WIN, _WIN)],
                    bufs[1 - b], sems[1 - b])

            nbase = cbase + w * _WIN

            @pl.loop(0, _WIN, step=_LANES, unroll=_UNROLL)
            def _vec(q):
                vlin = buf[pl.ds(q, _LANES)]
                idx = vlin - rbase
                nval = (nbase + q) + lane
                ok = (idx >= 0) & (idx < _G)
                idxc = jnp.clip(idx, 0, _G - 1)
                plsc.store_scatter(grid_v, [idxc], nval, mask=ok)
        return carry

    lax.fori_loop(0, _NWIN // 2, _outer, 0)
    pltpu.sync_copy(grid_v, grids_hbm.at[pl.ds(wid * _G, _G)])


@functools.cache
def _winner_kernel():
    return pl.kernel(
        _winner_body,
        out_type=jax.ShapeDtypeStruct((_NT * _G,), jnp.int32),
        mesh=_sc_mesh(),
        scratch_types=[pltpu.VMEM((_G,), jnp.int32),
                       pltpu.VMEM((_WIN,), jnp.int32),
                       pltpu.VMEM((_WIN,), jnp.int32),
                       pltpu.SemaphoreType.DMA,
                       pltpu.SemaphoreType.DMA],
        compiler_params=pltpu.CompilerParams(needs_layout_passes=False, use_tc_tiling_on_sc=True),
    )


def _compose_body(grids_hbm, colors_hbm, image_hbm, out_hbm,
                  acc_v, tmp_v, cidx0_v, cidx1_v, cidx2_v,
                  col0_v, col1_v, col2_v, img_v, sem):
    wid = lax.axis_index("s") * 2 + lax.axis_index("c")
    rp = wid >> 3
    off = (wid & 7) * _PIX_T
    pbase = wid * _PIX_T
    lane = lax.iota(jnp.int32, _LANES)

    pltpu.sync_copy(grids_hbm.at[pl.ds(rp * _G + off, _PIX_T)], acc_v)

    def _merge(k, carry):
        pltpu.sync_copy(grids_hbm.at[pl.ds((rp + 4 * k) * _G + off, _PIX_T)], tmp_v)

        @pl.loop(0, _PIX_T, step=_LANES, unroll=_UNROLL)
        def _mx(j):
            sl = pl.ds(j, _LANES)
            acc_v[sl] = jnp.maximum(acc_v[sl], tmp_v[sl])
        return carry

    lax.fori_loop(1, _NCHUNK, _merge, 0)

    cidx = (cidx0_v, cidx1_v, cidx2_v)
    cols = (col0_v, col1_v, col2_v)

    @pl.loop(0, _PIX_T, step=_LANES, unroll=_UNROLL)
    def _gidx(j):
        sl = pl.ds(j, _LANES)
        wv = acc_v[sl]
        dummy = (pbase + j) + lane
        safe = jnp.where(wv >= 0, wv, dummy)
        cidx0_v[sl] = safe
        cidx1_v[sl] = safe + _N
        cidx2_v[sl] = safe + 2 * _N

    copies = []
    for ch in range(3):
        for q in range(_PIX_T // 128):
            copies.append(pltpu.async_copy(
                colors_hbm.at[cidx[ch].at[pl.ds(q * 128, 128)]],
                cols[ch].at[pl.ds(q * 128, 128)],
                sem))
    for cp in copies:
        cp.wait()

    for ch in range(3):
        pltpu.sync_copy(image_hbm.at[pl.ds(ch * _HW + pbase, _PIX_T)], img_v)

        @pl.loop(0, _PIX_T, step=_LANES, unroll=_UNROLL)
        def _sel(j):
            sl = pl.ds(j, _LANES)
            m = acc_v[sl] >= 0
            cols[ch][sl] = jnp.where(m, cols[ch][sl], img_v[sl])
        pltpu.sync_copy(cols[ch], out_hbm.at[pl.ds(ch * _HW + pbase, _PIX_T)])


@functools.cache
def _compose_kernel():
    return pl.kernel(
        _compose_body,
        out_type=jax.ShapeDtypeStruct((3 * _HW,), jnp.float32),
        mesh=_sc_mesh(),
        scratch_types=[pltpu.VMEM((_PIX_T,), jnp.int32),
                       pltpu.VMEM((_PIX_T,), jnp.int32),
                       pltpu.VMEM((_PIX_T,), jnp.int32),
                       pltpu.VMEM((_PIX_T,), jnp.int32),
                       pltpu.VMEM((_PIX_T,), jnp.int32),
                       pltpu.VMEM((_PIX_T,), jnp.float32),
                       pltpu.VMEM((_PIX_T,), jnp.float32),
                       pltpu.VMEM((_PIX_T,), jnp.float32),
                       pltpu.VMEM((_PIX_T,), jnp.float32),
                       pltpu.SemaphoreType.DMA],
        compiler_params=pltpu.CompilerParams(needs_layout_passes=False, use_tc_tiling_on_sc=True),
    )


def kernel(image_grid, query_points, query_colors, intrinsics):
    xp = query_points[0, :, 0].reshape(_ROWS, 128)
    yp = query_points[0, :, 1].reshape(_ROWS, 128)
    zp = query_points[0, :, 2].reshape(_ROWS, 128)
    lin = _project(xp, yp, zp, intrinsics).reshape(_N)
    grids = _winner_kernel()(lin)
    colors_flat = query_colors.astype(jnp.float32).T.reshape(3 * _N)
    image_flat = image_grid.reshape(3 * _HW)
    out = _compose_kernel()(grids, colors_flat, image_flat)
    return out.reshape(1, 3, _H, _W)


# speedup vs baseline: 1.0002x; 1.0002x over previous
import functools

import jax
import jax.numpy as jnp
from jax import lax
from jax.experimental import pallas as pl
from jax.experimental.pallas import tpu as pltpu
from jax.experimental.pallas import tpu_sc as plsc

_EPS = 1e-07
_N = 2097152
_H = 512
_W = 512
_HW = _H * _W
_NT = 32
_NREG = 4
_G = _HW // _NREG
_NCHUNK = _NT // _NREG
_CHUNK = _N // _NCHUNK
_WIN = 4096
_NWIN = _CHUNK // _WIN
_PIX_T = _HW // _NT
_LANES = 16

_ROWS = _N // 128
_BLK = 1024


def _proj_body(x_ref, y_ref, z_ref, k_ref, lin_ref):
    x = x_ref[...]
    y = y_ref[...]
    z = z_ref[...]
    u = x * k_ref[0, 0] + y * k_ref[0, 1] + z * k_ref[0, 2]
    v = x * k_ref[1, 0] + y * k_ref[1, 1] + z * k_ref[1, 2]
    w = x * k_ref[2, 0] + y * k_ref[2, 1] + z * k_ref[2, 2]
    d = w + _EPS
    gx = u / d
    gy = v / d
    px = jnp.clip(jnp.floor(gx - 0.5), 0.0, float(_W - 1)).astype(jnp.int32)
    py = jnp.clip(jnp.floor(gy - 0.5), 0.0, float(_H - 1)).astype(jnp.int32)
    lin_ref[...] = py * _W + px


def _project(xp, yp, zp, intrinsics):
    spec = pl.BlockSpec((_BLK, 128), lambda i: (i, 0))
    return pl.pallas_call(
        _proj_body,
        grid=(_ROWS // _BLK,),
        in_specs=[spec, spec, spec,
                  pl.BlockSpec(memory_space=pltpu.SMEM)],
        out_specs=spec,
        out_shape=jax.ShapeDtypeStruct((_ROWS, 128), jnp.int32),
    )(xp, yp, zp, intrinsics)


@functools.cache
def _sc_mesh():
    return plsc.VectorSubcoreMesh(core_axis_name="c", subcore_axis_name="s",
                                  num_cores=2, num_subcores=16)


_UNROLL = 8


def _winner_body(lin_hbm, grids_hbm, grid_v, buf0_v, buf1_v, sem0, sem1):
    wid = lax.axis_index("s") * 2 + lax.axis_index("c")
    region = wid & 3
    chunk = wid >> 2
    lane = lax.iota(jnp.int32, _LANES)
    minus1 = jnp.full((_LANES,), -1, jnp.int32)

    @pl.loop(0, _G, step=_LANES, unroll=_UNROLL)
    def _init(i):
        grid_v[pl.ds(i, _LANES)] = minus1

    rbase = region * _G
    cbase = chunk * _CHUNK
    bufs = (buf0_v, buf1_v)
    sems = (sem0, sem1)

    pltpu.async_copy(lin_hbm.at[pl.ds(cbase, _WIN)], buf0_v, sem0)

    def _outer(g, carry):
        for b in range(2):
            w = 2 * g + b
            buf = bufs[b]
            pltpu.make_async_copy(
                lin_hbm.at[pl.ds(0, _WIN)], buf, sems[b]).wait()

            @pl.when(w < _NWIN - 1)
            def _start_next():
                pltpu.async_copy(
                    lin_hbm.at[pl.ds(cbase + (w + 1) * _WIN, _WIN)],
                    bufs[1 - b], sems[1 - b])

            nbase = cbase + w * _WIN

            @pl.loop(0, _WIN, step=_LANES, unroll=_UNROLL)
            def _vec(q):
                vlin = buf[pl.ds(q, _LANES)]
                idx = vlin - rbase
                nval = (nbase + q) + lane
                _, last = plsc.scan_count(idx)
                ok = last & (idx >= 0) & (idx < _G)
                idxc = jnp.clip(idx, 0, _G - 1)
                plsc.store_scatter(grid_v, [idxc], nval, mask=ok)
        return carry

    lax.fori_loop(0, _NWIN // 2, _outer, 0)
    pltpu.sync_copy(grid_v, grids_hbm.at[pl.ds(wid * _G, _G)])


@functools.cache
def _winner_kernel():
    return pl.kernel(
        _winner_body,
        out_type=jax.ShapeDtypeStruct((_NT * _G,), jnp.int32),
        mesh=_sc_mesh(),
        scratch_types=[pltpu.VMEM((_G,), jnp.int32),
                       pltpu.VMEM((_WIN,), jnp.int32),
                       pltpu.VMEM((_WIN,), jnp.int32),
                       pltpu.SemaphoreType.DMA,
                       pltpu.SemaphoreType.DMA],
        compiler_params=pltpu.CompilerParams(needs_layout_passes=False, use_tc_tiling_on_sc=True),
    )


def _compose_body(grids_hbm, colors_hbm, image_hbm, out_hbm,
                  acc_v, tmp_v, cidx0_v, cidx1_v, cidx2_v,
                  col0_v, col1_v, col2_v, img_v, sem):
    wid = lax.axis_index("s") * 2 + lax.axis_index("c")
    rp = wid >> 3
    off = (wid & 7) * _PIX_T
    pbase = wid * _PIX_T
    lane = lax.iota(jnp.int32, _LANES)

    pltpu.sync_copy(grids_hbm.at[pl.ds(rp * _G + off, _PIX_T)], acc_v)

    def _merge(k, carry):
        pltpu.sync_copy(grids_hbm.at[pl.ds((rp + 4 * k) * _G + off, _PIX_T)], tmp_v)

        @pl.loop(0, _PIX_T, step=_LANES, unroll=_UNROLL)
        def _mx(j):
            sl = pl.ds(j, _LANES)
            acc_v[sl] = jnp.maximum(acc_v[sl], tmp_v[sl])
        return carry

    lax.fori_loop(1, _NCHUNK, _merge, 0)

    cidx = (cidx0_v, cidx1_v, cidx2_v)
    cols = (col0_v, col1_v, col2_v)

    @pl.loop(0, _PIX_T, step=_LANES, unroll=_UNROLL)
    def _gidx(j):
        sl = pl.ds(j, _LANES)
        wv = acc_v[sl]
        dummy = (pbase + j) + lane
        safe = jnp.where(wv >= 0, wv, dummy)
        cidx0_v[sl] = safe
        cidx1_v[sl] = safe + _N
        cidx2_v[sl] = safe + 2 * _N

    copies = []
    for ch in range(3):
        for q in range(_PIX_T // 128):
            copies.append(pltpu.async_copy(
                colors_hbm.at[cidx[ch].at[pl.ds(q * 128, 128)]],
                cols[ch].at[pl.ds(q * 128, 128)],
                sem))
    for cp in copies:
        cp.wait()

    for ch in range(3):
        pltpu.sync_copy(image_hbm.at[pl.ds(ch * _HW + pbase, _PIX_T)], img_v)

        @pl.loop(0, _PIX_T, step=_LANES, unroll=_UNROLL)
        def _sel(j):
            sl = pl.ds(j, _LANES)
            m = acc_v[sl] >= 0
            cols[ch][sl] = jnp.where(m, cols[ch][sl], img_v[sl])
        pltpu.sync_copy(cols[ch], out_hbm.at[pl.ds(ch * _HW + pbase, _PIX_T)])


@functools.cache
def _compose_kernel():
    return pl.kernel(
        _compose_body,
        out_type=jax.ShapeDtypeStruct((3 * _HW,), jnp.float32),
        mesh=_sc_mesh(),
        scratch_types=[pltpu.VMEM((_PIX_T,), jnp.int32),
                       pltpu.VMEM((_PIX_T,), jnp.int32),
                       pltpu.VMEM((_PIX_T,), jnp.int32),
                       pltpu.VMEM((_PIX_T,), jnp.int32),
                       pltpu.VMEM((_PIX_T,), jnp.int32),
                       pltpu.VMEM((_PIX_T,), jnp.float32),
                       pltpu.VMEM((_PIX_T,), jnp.float32),
                       pltpu.VMEM((_PIX_T,), jnp.float32),
                       pltpu.VMEM((_PIX_T,), jnp.float32),
                       pltpu.SemaphoreType.DMA],
        compiler_params=pltpu.CompilerParams(needs_layout_passes=False, use_tc_tiling_on_sc=True),
    )


def kernel(image_grid, query_points, query_colors, intrinsics):
    xp = query_points[0, :, 0].reshape(_ROWS, 128)
    yp = query_points[0, :, 1].reshape(_ROWS, 128)
    zp = query_points[0, :, 2].reshape(_ROWS, 128)
    lin = _project(xp, yp, zp, intrinsics).reshape(_N)
    grids = _winner_kernel()(lin)
    colors_flat = query_colors.astype(jnp.float32).T.reshape(3 * _N)
    image_flat = image_grid.reshape(3 * _HW)
    out = _compose_kernel()(grids, colors_flat, image_flat)
    return out.reshape(1, 3, _H, _W)


# speedup vs baseline: 2.0349x; 2.0346x over previous
import functools

import jax
import jax.numpy as jnp
from jax import lax
from jax.experimental import pallas as pl
from jax.experimental.pallas import tpu as pltpu
from jax.experimental.pallas import tpu_sc as plsc

_EPS = 1e-07
_N = 2097152
_H = 512
_W = 512
_HW = _H * _W
_NT = 32
_NREG = 4
_G = _HW // _NREG
_NCHUNK = _NT // _NREG
_CHUNK = _N // _NCHUNK
_WIN = 4096
_NWIN = _CHUNK // _WIN
_PIX_T = _HW // _NT
_LANES = 16

_ROWS = _N // 128
_BLK = 1024


def _proj_body(x_ref, y_ref, z_ref, k_ref, lin_ref):
    x = x_ref[...]
    y = y_ref[...]
    z = z_ref[...]
    u = x * k_ref[0, 0] + y * k_ref[0, 1] + z * k_ref[0, 2]
    v = x * k_ref[1, 0] + y * k_ref[1, 1] + z * k_ref[1, 2]
    w = x * k_ref[2, 0] + y * k_ref[2, 1] + z * k_ref[2, 2]
    d = w + _EPS
    gx = u / d
    gy = v / d
    px = jnp.clip(jnp.floor(gx - 0.5), 0.0, float(_W - 1)).astype(jnp.int32)
    py = jnp.clip(jnp.floor(gy - 0.5), 0.0, float(_H - 1)).astype(jnp.int32)
    lin_ref[...] = py * _W + px


def _project(xp, yp, zp, intrinsics):
    spec = pl.BlockSpec((_BLK, 128), lambda i: (i, 0))
    return pl.pallas_call(
        _proj_body,
        grid=(_ROWS // _BLK,),
        in_specs=[spec, spec, spec,
                  pl.BlockSpec(memory_space=pltpu.SMEM)],
        out_specs=spec,
        out_shape=jax.ShapeDtypeStruct((_ROWS, 128), jnp.int32),
    )(xp, yp, zp, intrinsics)


@functools.cache
def _sc_mesh():
    return plsc.VectorSubcoreMesh(core_axis_name="c", subcore_axis_name="s",
                                  num_cores=2, num_subcores=16)


_UNROLL = 8


def _winner_body(lin_hbm, grids_hbm, grid_v, buf0_v, buf1_v, sem0, sem1):
    wid = lax.axis_index("s") * 2 + lax.axis_index("c")
    region = wid & 3
    chunk = wid >> 2
    lane = lax.iota(jnp.int32, _LANES)
    minus1 = jnp.full((_LANES,), -1, jnp.int32)

    @pl.loop(0, _G, step=_LANES, unroll=_UNROLL)
    def _init(i):
        grid_v[pl.ds(i, _LANES)] = minus1

    rbase = region * _G
    cbase = chunk * _CHUNK
    bufs = (buf0_v, buf1_v)
    sems = (sem0, sem1)

    pltpu.async_copy(lin_hbm.at[pl.ds(cbase, _WIN)], buf0_v, sem0)

    def _outer(g, carry):
        for b in range(2):
            w = 2 * g + b
            buf = bufs[b]
            pltpu.make_async_copy(
                lin_hbm.at[pl.ds(0, _WIN)], buf, sems[b]).wait()

            @pl.when(w < _NWIN - 1)
            def _start_next():
                pltpu.async_copy(
                    lin_hbm.at[pl.ds(cbase + (w + 1) * _WIN, _WIN)],
                    bufs[1 - b], sems[1 - b])

            nbase = cbase + w * _WIN

            @pl.loop(0, _WIN, step=_LANES, unroll=_UNROLL)
            def _vec(q):
                vlin = buf[pl.ds(q, _LANES)]
                idx = vlin - rbase
                nval = (nbase + q) + lane
                ok = (idx >= 0) & (idx < _G)
                idxc = jnp.clip(idx, 0, _G - 1)
                plsc.store_scatter(grid_v, [idxc], nval, mask=ok)
        return carry

    lax.fori_loop(0, _NWIN // 2, _outer, 0)
    pltpu.sync_copy(grid_v, grids_hbm.at[pl.ds(wid * _G, _G)])


@functools.cache
def _winner_kernel():
    return pl.kernel(
        _winner_body,
        out_type=jax.ShapeDtypeStruct((_NT * _G,), jnp.int32),
        mesh=_sc_mesh(),
        scratch_types=[pltpu.VMEM((_G,), jnp.int32),
                       pltpu.VMEM((_WIN,), jnp.int32),
                       pltpu.VMEM((_WIN,), jnp.int32),
                       pltpu.SemaphoreType.DMA,
                       pltpu.SemaphoreType.DMA],
        compiler_params=pltpu.CompilerParams(needs_layout_passes=False, use_tc_tiling_on_sc=True),
    )


def _compose_body(grids_hbm, c0_hbm, c1_hbm, c2_hbm, image_hbm, out_hbm,
                  acc_v, tmp_v, cidx0_v, cidx1_v, cidx2_v,
                  col0_v, col1_v, col2_v, img_v, sem):
    wid = lax.axis_index("s") * 2 + lax.axis_index("c")
    rp = wid >> 3
    off = (wid & 7) * _PIX_T
    pbase = wid * _PIX_T
    lane = lax.iota(jnp.int32, _LANES)

    pltpu.sync_copy(grids_hbm.at[pl.ds(rp * _G + off, _PIX_T)], acc_v)

    def _merge(k, carry):
        pltpu.sync_copy(grids_hbm.at[pl.ds((rp + 4 * k) * _G + off, _PIX_T)], tmp_v)

        @pl.loop(0, _PIX_T, step=_LANES, unroll=_UNROLL)
        def _mx(j):
            sl = pl.ds(j, _LANES)
            acc_v[sl] = jnp.maximum(acc_v[sl], tmp_v[sl])
        return carry

    lax.fori_loop(1, _NCHUNK, _merge, 0)

    cidx = (cidx0_v, cidx1_v, cidx2_v)
    cols = (col0_v, col1_v, col2_v)

    @pl.loop(0, _PIX_T, step=_LANES, unroll=_UNROLL)
    def _gidx(j):
        sl = pl.ds(j, _LANES)
        wv = acc_v[sl]
        dummy = (pbase + j) + lane
        safe = jnp.where(wv >= 0, wv, dummy)
        cidx0_v[sl] = safe
        cidx1_v[sl] = safe
        cidx2_v[sl] = safe

    chbm = (c0_hbm, c1_hbm, c2_hbm)
    copies = []
    for ch in range(3):
        for q in range(_PIX_T // 128):
            copies.append(pltpu.async_copy(
                chbm[ch].at[cidx[ch].at[pl.ds(q * 128, 128)]],
                cols[ch].at[pl.ds(q * 128, 128)],
                sem))
    for cp in copies:
        cp.wait()

    for ch in range(3):
        pltpu.sync_copy(image_hbm.at[pl.ds(ch * _HW + pbase, _PIX_T)], img_v)

        @pl.loop(0, _PIX_T, step=_LANES, unroll=_UNROLL)
        def _sel(j):
            sl = pl.ds(j, _LANES)
            m = acc_v[sl] >= 0
            cols[ch][sl] = jnp.where(m, cols[ch][sl], img_v[sl])
        pltpu.sync_copy(cols[ch], out_hbm.at[pl.ds(ch * _HW + pbase, _PIX_T)])


@functools.cache
def _compose_kernel():
    return pl.kernel(
        _compose_body,
        out_type=jax.ShapeDtypeStruct((3 * _HW,), jnp.float32),
        mesh=_sc_mesh(),
        scratch_types=[pltpu.VMEM((_PIX_T,), jnp.int32),
                       pltpu.VMEM((_PIX_T,), jnp.int32),
                       pltpu.VMEM((_PIX_T,), jnp.int32),
                       pltpu.VMEM((_PIX_T,), jnp.int32),
                       pltpu.VMEM((_PIX_T,), jnp.int32),
                       pltpu.VMEM((_PIX_T,), jnp.float32),
                       pltpu.VMEM((_PIX_T,), jnp.float32),
                       pltpu.VMEM((_PIX_T,), jnp.float32),
                       pltpu.VMEM((_PIX_T,), jnp.float32),
                       pltpu.SemaphoreType.DMA],
        compiler_params=pltpu.CompilerParams(needs_layout_passes=False, use_tc_tiling_on_sc=True),
    )


def kernel(image_grid, query_points, query_colors, intrinsics):
    xp = query_points[0, :, 0].reshape(_ROWS, 128)
    yp = query_points[0, :, 1].reshape(_ROWS, 128)
    zp = query_points[0, :, 2].reshape(_ROWS, 128)
    lin = _project(xp, yp, zp, intrinsics).reshape(_N)
    grids = _winner_kernel()(lin)
    c0 = query_colors[:, 0]
    c1 = query_colors[:, 1]
    c2 = query_colors[:, 2]
    image_flat = image_grid.reshape(3 * _HW)
    out = _compose_kernel()(grids, c0, c1, c2, image_flat)
    return out.reshape(1, 3, _H, _W)
